# baseline (speedup 1.0000x reference)
import functools
import jax
import jax.numpy as jnp
from jax import lax
from jax.experimental import pallas as pl
from jax.experimental.pallas import tpu as pltpu
from jax.experimental.pallas import tpu_sc as plsc

B, N, D = 4096, 128, 64
ROW = N * D
NC, NS = 2, 16
NW = NC * NS
SC_B = 1536
BPW = SC_B // NW
CH = 2
NCHUNK = BPW // CH
CHW = CH * ROW
TC_ROWS = (B - SC_B) * D
BLKR = 1024
WEIGHT = 0.05


def _sc_body(x_hbm, out_hbm, buf0, buf1, acc_v, sem0, sem1):
    wid = lax.axis_index("s") * NC + lax.axis_index("c")
    base = wid * (BPW * ROW)
    zero = jnp.zeros((16,), jnp.float32)
    mask = jnp.where(lax.iota(jnp.int32, 16) < 15, 1.0, 0.0).astype(jnp.float32)

    buf0[pl.ds(CHW, 16)] = zero
    buf1[pl.ds(CHW, 16)] = zero

    def start(c, buf, sem):
        pltpu.async_copy(x_hbm.at[pl.ds(base + c * CHW, CHW)],
                         buf.at[pl.ds(0, CHW)], sem)

    def wait(buf, sem):
        pltpu.make_async_copy(x_hbm.at[pl.ds(base, CHW)],
                              buf.at[pl.ds(0, CHW)], sem).wait()

    def compute(buf, accs):
        @plsc.parallel_loop(0, CHW, N, unroll=4, carry=accs)
        def accs(o, accs):
            new = list(accs)
            for j in range(8):
                va = buf[pl.ds(o + 16 * j, 16)]
                vs = buf[pl.ds(o + 16 * j + 1, 16)]
                d = va - vs
                if j == 7:
                    d = d * mask
                new[j] = new[j] + d * d
            return tuple(new)
        return accs

    start(0, buf0, sem0)

    def outer(g, accs):
        c0 = 2 * g
        wait(buf0, sem0)
        start(c0 + 1, buf1, sem1)
        accs = compute(buf0, accs)
        wait(buf1, sem1)

        @pl.when(c0 + 2 < NCHUNK)
        def _():
            start(c0 + 2, buf0, sem0)

        return compute(buf1, accs)

    accs = lax.fori_loop(0, NCHUNK // 2, outer, (zero,) * 8)
    acc_v[...] = ((accs[0] + accs[1]) + (accs[2] + accs[3])) + \
                 ((accs[4] + accs[5]) + (accs[6] + accs[7]))
    pltpu.sync_copy(acc_v, out_hbm.at[wid])


def _tc_body(x_ref, out_ref):
    i = pl.program_id(0)

    @pl.when(i == 0)
    def _():
        out_ref[0, 0] = 0.0

    x = x_ref[...]
    d = x[:, :-1] - x[:, 1:]
    out_ref[0, 0] += jnp.sum(d * d)


@jax.jit
def kernel(coords):
    xt = coords.transpose(0, 2, 1)
    x = xt.reshape(B * ROW)
    x2 = xt.reshape(B * D, N)

    mesh = plsc.VectorSubcoreMesh(core_axis_name="c", subcore_axis_name="s",
                                  num_cores=NC, num_subcores=NS)
    sc_partials = pl.kernel(
        _sc_body,
        out_type=jax.ShapeDtypeStruct((NW, 16), jnp.float32),
        mesh=mesh,
        scratch_types=[
            pltpu.VMEM((CHW + 16,), jnp.float32),
            pltpu.VMEM((CHW + 16,), jnp.float32),
            pltpu.VMEM((16,), jnp.float32),
            pltpu.SemaphoreType.DMA,
            pltpu.SemaphoreType.DMA,
        ],
    )(x)

    row0 = SC_B * D // BLKR
    tc_sum = pl.pallas_call(
        _tc_body,
        out_shape=jax.ShapeDtypeStruct((1, 1), jnp.float32),
        grid=(TC_ROWS // BLKR,),
        in_specs=[pl.BlockSpec((BLKR, N), lambda i: (row0 + i, 0))],
        out_specs=pl.BlockSpec(memory_space=pltpu.SMEM),
    )(x2)

    total = jnp.sum(sc_partials) + tc_sum[0, 0]
    return (WEIGHT / (B * (N - 1))) * total
